# Initial kernel scaffold; baseline (speedup 1.0000x reference)
import jax
import jax.numpy as jnp
from jax.experimental import pallas as pl


def kernel(node_fea, node_one_hot, edge_sh, edge_fea, edge_length_embedded, edge_index, batch, selfloop_edge, edge_length, W_pre, b_pre, W_tp, W1, b1, W2, b2, W3, b3, W_post, b_post, W_sc, gamma, beta):
    raise NotImplementedError("write your pallas kernel here")


# speedup vs baseline: 1.7582x; 1.7582x over previous
import functools

import jax
import jax.numpy as jnp
import numpy as np
from jax import lax
from jax.experimental import pallas as pl
from jax.experimental.pallas import tpu as pltpu
from jax.experimental.pallas import tpu_sc as plsc

N = 10000
E = 320000
D = 128
DE = 128
S = 4
FC = 64
SH = 1
IN1 = D + D + DE

NC = 2
NS = 16
EPT = E // (NC * NS)
B = 80
NBLK = EPT // B
NCHUNK = N // B

_INV_S = float(1.0 / np.sqrt(IN1 * SH))
_SC_SCALE = float(1.0 / np.sqrt(D * S))


def _node_prep_body(nf_ref, oh_ref, wpre_ref, bpre_ref, wa_ref, wb_ref,
                    wsc_ref, u_ref, v_ref, sc_ref):
    nf = nf_ref[...]
    x = jnp.dot(nf, wpre_ref[...], preferred_element_type=jnp.float32) + bpre_ref[...]
    u_ref[...] = jnp.dot(x, wa_ref[...], preferred_element_type=jnp.float32) * _INV_S
    v_ref[...] = jnp.dot(x, wb_ref[...], preferred_element_type=jnp.float32) * _INV_S
    oh = oh_ref[...]
    acc = jnp.zeros_like(nf)
    for s in range(S):
        acc += jnp.dot(nf * oh[:, s:s + 1], wsc_ref[s],
                       preferred_element_type=jnp.float32)
    sc_ref[...] = acc * _SC_SCALE


def _node_prep(node_fea, node_one_hot, w_pre, b_pre, wa, wb, wsc_t):
    blk = 2000
    grid = N // blk
    return pl.pallas_call(
        _node_prep_body,
        grid=(grid,),
        in_specs=[
            pl.BlockSpec((blk, D), lambda i: (i, 0)),
            pl.BlockSpec((blk, S), lambda i: (i, 0)),
            pl.BlockSpec((D, D), lambda i: (0, 0)),
            pl.BlockSpec((1, D), lambda i: (0, 0)),
            pl.BlockSpec((D, D), lambda i: (0, 0)),
            pl.BlockSpec((D, D), lambda i: (0, 0)),
            pl.BlockSpec((S, D, D), lambda i: (0, 0, 0)),
        ],
        out_specs=[
            pl.BlockSpec((blk, D), lambda i: (i, 0)),
            pl.BlockSpec((blk, D), lambda i: (i, 0)),
            pl.BlockSpec((blk, D), lambda i: (i, 0)),
        ],
        out_shape=[
            jax.ShapeDtypeStruct((N, D), jnp.float32),
            jax.ShapeDtypeStruct((N, D), jnp.float32),
            jax.ShapeDtypeStruct((N, D), jnp.float32),
        ],
    )(node_fea, node_one_hot, w_pre, b_pre.reshape(1, D), wa, wb, wsc_t)


def _edge_prep_body(ef_ref, ele_ref, wc_ref, w1_ref, b1_ref, w2_ref, b2_ref,
                    w3_ref, b3_ref, c_ref, w_ref):
    c_ref[...] = jnp.dot(ef_ref[...], wc_ref[...],
                         preferred_element_type=jnp.float32) * _INV_S
    h = jax.nn.silu(jnp.dot(ele_ref[...], w1_ref[...],
                            preferred_element_type=jnp.float32) + b1_ref[...])
    h = jax.nn.silu(jnp.dot(h, w2_ref[...],
                            preferred_element_type=jnp.float32) + b2_ref[...])
    w_ref[...] = jnp.dot(h, w3_ref[...],
                         preferred_element_type=jnp.float32) + b3_ref[...]


def _edge_prep(edge_fea, ele, wc, w1, b1, w2, b2, w3, b3):
    blk = 4000
    grid = E // blk
    return pl.pallas_call(
        _edge_prep_body,
        grid=(grid,),
        in_specs=[
            pl.BlockSpec((blk, DE), lambda i: (i, 0)),
            pl.BlockSpec((blk, FC), lambda i: (i, 0)),
            pl.BlockSpec((DE, D), lambda i: (0, 0)),
            pl.BlockSpec((FC, 64), lambda i: (0, 0)),
            pl.BlockSpec((1, 64), lambda i: (0, 0)),
            pl.BlockSpec((64, 64), lambda i: (0, 0)),
            pl.BlockSpec((1, 64), lambda i: (0, 0)),
            pl.BlockSpec((64, D), lambda i: (0, 0)),
            pl.BlockSpec((1, D), lambda i: (0, 0)),
        ],
        out_specs=[
            pl.BlockSpec((blk, D), lambda i: (i, 0)),
            pl.BlockSpec((blk, D), lambda i: (i, 0)),
        ],
        out_shape=[
            jax.ShapeDtypeStruct((E, D), jnp.float32),
            jax.ShapeDtypeStruct((E, D), jnp.float32),
        ],
    )(edge_fea, ele, wc, w1, b1.reshape(1, 64), w2, b2.reshape(1, 64),
      w3, b3.reshape(1, D))


def _sc_body(u_hbm, v_hbm, ii_hbm, jj_hbm, c_hbm, w_hbm, sh_hbm, out_hbm,
             ii_v, jj_v, gi_v, gj_v, c_v, w_v, sh_v, acc, sem1, sem2):
    cid = lax.axis_index("c")
    sid = lax.axis_index("s")
    iota16 = lax.broadcasted_iota(jnp.int32, (16,), 0)
    zeros16 = jnp.zeros((16,), jnp.float32)

    def _zero_row(r, _):
        for q in range(D // 16):
            gi_v[r, pl.ds(q * 16, 16)] = zeros16
        return 0
    lax.fori_loop(0, B, _zero_row, 0)

    def _zero_chunk(k, _):
        chunk = sid + NS * k
        @pl.when(chunk < NCHUNK)
        def _():
            pltpu.sync_copy(gi_v, acc.at[pl.ds(chunk * B, B)])
        return 0
    lax.fori_loop(0, (NCHUNK + NS - 1) // NS, _zero_chunk, 0)

    plsc.subcore_barrier()

    base = cid * (E // NC) + sid * EPT

    def _block(k, _):
        start = base + k * B
        pltpu.sync_copy(ii_hbm.at[pl.ds(start, B)], ii_v)
        pltpu.sync_copy(jj_hbm.at[pl.ds(start, B)], jj_v)
        d1 = pltpu.async_copy(u_hbm.at[ii_v], gi_v, sem1)
        d2 = pltpu.async_copy(v_hbm.at[jj_v], gj_v, sem2)
        pltpu.sync_copy(c_hbm.at[pl.ds(start, B)], c_v)
        pltpu.sync_copy(w_hbm.at[pl.ds(start, B)], w_v)
        pltpu.sync_copy(sh_hbm.at[pl.ds(start, B)], sh_v)
        d1.wait()
        d2.wait()

        def _group(g, _):
            shv = sh_v[pl.ds(g * 16, 16)]
            for j in range(16):
                s = shv[j]
                r = g * 16 + j
                for q in range(D // 16):
                    sl = pl.ds(q * 16, 16)
                    z = (gi_v[r, sl] + gj_v[r, sl] + c_v[r, sl]) * s
                    m = z / (1.0 + jnp.exp(-z)) * w_v[r, sl]
                    gi_v[r, sl] = m
            return 0
        lax.fori_loop(0, B // 16, _group, 0)

        pltpu.sync_copy(gi_v, acc.at[ii_v], add=True)
        return 0
    lax.fori_loop(0, NBLK, _block, 0)

    plsc.subcore_barrier()

    def _dump_chunk(k, _):
        chunk = sid + NS * k
        @pl.when(chunk < NCHUNK)
        def _():
            pltpu.sync_copy(acc.at[pl.ds(chunk * B, B)],
                            out_hbm.at[pl.ds(cid * N + chunk * B, B)])
        return 0
    lax.fori_loop(0, (NCHUNK + NS - 1) // NS, _dump_chunk, 0)


def _sc_aggregate(u, v, ii, jj, c, w, sh):
    mesh = plsc.VectorSubcoreMesh(core_axis_name="c", subcore_axis_name="s",
                                  num_cores=NC, num_subcores=NS)
    f = pl.kernel(
        _sc_body,
        out_type=jax.ShapeDtypeStruct((NC * N, D), jnp.float32),
        mesh=mesh,
        scratch_types=[
            pltpu.VMEM((B,), jnp.int32),
            pltpu.VMEM((B,), jnp.int32),
            pltpu.VMEM((B, D), jnp.float32),
            pltpu.VMEM((B, D), jnp.float32),
            pltpu.VMEM((B, D), jnp.float32),
            pltpu.VMEM((B, D), jnp.float32),
            pltpu.VMEM((B,), jnp.float32),
            pltpu.VMEM_SHARED((N, D), jnp.float32),
            pltpu.SemaphoreType.DMA,
            pltpu.SemaphoreType.DMA,
        ],
    )
    return f(u, v, ii, jj, c, w, sh)


def _epilogue_body(aggp_ref, sc_ref, nf_ref, wpost_ref, bpost_ref,
                   gamma_ref, beta_ref, out_ref):
    agg = aggp_ref[0] + aggp_ref[1]
    o = jnp.dot(agg, wpost_ref[...], preferred_element_type=jnp.float32)
    o = o + bpost_ref[...] + sc_ref[...]
    m_d = jnp.mean(o, axis=0, keepdims=True)
    s_d = jnp.mean(o * o, axis=0, keepdims=True)
    rms = jnp.mean(s_d - m_d * m_d)
    inv = lax.rsqrt(rms + 1e-5)
    out_ref[...] = ((o - m_d) * inv * gamma_ref[...] + beta_ref[...]
                    + nf_ref[...])


def _epilogue(aggp, sc, node_fea, w_post, b_post, gamma, beta):
    return pl.pallas_call(
        _epilogue_body,
        out_shape=jax.ShapeDtypeStruct((N, D), jnp.float32),
    )(aggp.reshape(NC, N, D), sc, node_fea, w_post, b_post.reshape(1, D),
      gamma.reshape(1, D), beta.reshape(1, D))


def kernel(node_fea, node_one_hot, edge_sh, edge_fea, edge_length_embedded,
           edge_index, batch, selfloop_edge, edge_length,
           W_pre, b_pre, W_tp, W1, b1, W2, b2, W3, b3, W_post, b_post,
           W_sc, gamma, beta):
    w_flat = W_tp.reshape(IN1, D)
    wa = w_flat[:D]
    wb = w_flat[D:2 * D]
    wc = w_flat[2 * D:]
    wsc_t = W_sc.transpose(1, 0, 2)

    u, v, sc = _node_prep(node_fea, node_one_hot, W_pre, b_pre, wa, wb, wsc_t)
    c, w = _edge_prep(edge_fea, edge_length_embedded, wc, W1, b1, W2, b2, W3, b3)

    ii = edge_index[0].astype(jnp.int32)
    jj = edge_index[1].astype(jnp.int32)
    sh = edge_sh.reshape(E)

    aggp = _sc_aggregate(u, v, ii, jj, c, w, sh)
    return _epilogue(aggp, sc, node_fea, W_post, b_post, gamma, beta)
